# baseline (speedup 1.0000x reference)
import functools

import jax
import jax.numpy as jnp
from jax.experimental import pallas as pl

_N = 32768
_D = 768
_T = 8
_BLOCK = 2048


def _tgate_kernel(x_ref, w_ref, b_ref, a_ref, o_ref):
    acc = jnp.dot(x_ref[...], w_ref[...], preferred_element_type=jnp.float32)
    accT = acc.T + b_ref[...]

    e = jnp.exp(accT[0:2 * _T, :])
    ec = e[0:_T, :]
    sig = 1.0 / (1.0 + e[_T:2 * _T, :])
    s = accT[2 * _T:3 * _T, :]

    m1 = jnp.max(s, axis=0, keepdims=True)
    mask1 = s == m1
    f1 = mask1.astype(jnp.float32)
    s2 = jnp.where(mask1, -jnp.inf, s)
    m2 = jnp.max(s2, axis=0, keepdims=True)
    f2 = (s2 == m2).astype(jnp.float32)
    cnt1 = jnp.sum(f1, axis=0, keepdims=True)
    cnt2 = jnp.sum(f2, axis=0, keepdims=True)
    p1 = 1.0 / (1.0 + jnp.exp(m2 - m1))
    sparse = jnp.where(cnt1 > 1.0, f1 / cnt1,
                       p1 * f1 + (1.0 - p1) * (f2 / cnt2))

    soft = ec / jnp.sum(ec, axis=0, keepdims=True)
    gsum = jnp.sum(sig, axis=0, keepdims=True)

    a = a_ref[0, 0]
    outT = (a * sparse + (1.0 - a) * soft) * gsum
    o_ref[...] = outT.T


@functools.partial(jax.jit, static_argnames=())
def _tgate(x, w, b, a):
    grid = (_N // _BLOCK,)
    return pl.pallas_call(
        _tgate_kernel,
        grid=grid,
        in_specs=[
            pl.BlockSpec((_BLOCK, _D), lambda i: (i, 0)),
            pl.BlockSpec((_D, 3 * _T), lambda i: (0, 0)),
            pl.BlockSpec((3 * _T, 1), lambda i: (0, 0)),
            pl.BlockSpec((1, 1), lambda i: (0, 0)),
        ],
        out_specs=pl.BlockSpec((_BLOCK, _T), lambda i: (i, 0)),
        out_shape=jax.ShapeDtypeStruct((_N, _T), jnp.float32),
    )(x, w, b, a)


def kernel(x, Wc, bc, Ws, bs, Wg, bg, alpha):
    w = jnp.concatenate([Wc, -Wg, Ws], axis=1)
    b = jnp.concatenate([bc, -bg, bs], axis=0).reshape(3 * _T, 1)
    a = jax.nn.sigmoid(alpha).reshape(1, 1)
    return _tgate(x, w, b, a)


# speedup vs baseline: 1.0043x; 1.0043x over previous
import functools

import jax
import jax.numpy as jnp
from jax.experimental import pallas as pl

_N = 32768
_D = 768
_T = 8
_BLOCK = 8192


def _tgate_kernel(x_ref, w_ref, b_ref, a_ref, o_ref):
    acc = jnp.dot(x_ref[...], w_ref[...], preferred_element_type=jnp.float32)
    accT = acc.T + b_ref[...]

    e = jnp.exp(accT[0:2 * _T, :])
    ec = e[0:_T, :]
    sig = 1.0 / (1.0 + e[_T:2 * _T, :])
    s = accT[2 * _T:3 * _T, :]

    m1 = jnp.max(s, axis=0, keepdims=True)
    mask1 = s == m1
    f1 = mask1.astype(jnp.float32)
    s2 = jnp.where(mask1, -jnp.inf, s)
    m2 = jnp.max(s2, axis=0, keepdims=True)
    f2 = (s2 == m2).astype(jnp.float32)
    cnt1 = jnp.sum(f1, axis=0, keepdims=True)
    cnt2 = jnp.sum(f2, axis=0, keepdims=True)
    p1 = 1.0 / (1.0 + jnp.exp(m2 - m1))
    sparse = jnp.where(cnt1 > 1.0, f1 / cnt1,
                       p1 * f1 + (1.0 - p1) * (f2 / cnt2))

    soft = ec / jnp.sum(ec, axis=0, keepdims=True)
    gsum = jnp.sum(sig, axis=0, keepdims=True)

    a = a_ref[0, 0]
    outT = (a * sparse + (1.0 - a) * soft) * gsum
    o_ref[...] = outT.T


@functools.partial(jax.jit, static_argnames=())
def _tgate(x, w, b, a):
    grid = (_N // _BLOCK,)
    return pl.pallas_call(
        _tgate_kernel,
        grid=grid,
        in_specs=[
            pl.BlockSpec((_BLOCK, _D), lambda i: (i, 0)),
            pl.BlockSpec((_D, 3 * _T), lambda i: (0, 0)),
            pl.BlockSpec((3 * _T, 1), lambda i: (0, 0)),
            pl.BlockSpec((1, 1), lambda i: (0, 0)),
        ],
        out_specs=pl.BlockSpec((_BLOCK, _T), lambda i: (i, 0)),
        out_shape=jax.ShapeDtypeStruct((_N, _T), jnp.float32),
    )(x, w, b, a)


def kernel(x, Wc, bc, Ws, bs, Wg, bg, alpha):
    w = jnp.concatenate([Wc, -Wg, Ws], axis=1)
    b = jnp.concatenate([bc, -bg, bs], axis=0).reshape(3 * _T, 1)
    a = jax.nn.sigmoid(alpha).reshape(1, 1)
    return _tgate(x, w, b, a)


# speedup vs baseline: 1.0353x; 1.0309x over previous
import functools

import jax
import jax.numpy as jnp
from jax import lax
from jax.experimental import pallas as pl
from jax.experimental.pallas import tpu as pltpu

_N = 32768
_D = 768
_T = 8
_CHUNK = 2048
_NBUF = 4
_STEPS = _N // _CHUNK


def _epilogue(xb, w_ref, b_ref, a_ref, o_ref):
    acc = jnp.dot(xb, w_ref[...], preferred_element_type=jnp.float32)
    accT = acc.T + b_ref[...]

    e = jnp.exp(accT[0:2 * _T, :])
    ec = e[0:_T, :]
    sig = 1.0 / (1.0 + e[_T:2 * _T, :])
    s = accT[2 * _T:3 * _T, :]

    m1 = jnp.max(s, axis=0, keepdims=True)
    mask1 = s == m1
    f1 = mask1.astype(jnp.float32)
    s2 = jnp.where(mask1, -jnp.inf, s)
    m2 = jnp.max(s2, axis=0, keepdims=True)
    f2 = (s2 == m2).astype(jnp.float32)
    cnt1 = jnp.sum(f1, axis=0, keepdims=True)
    cnt2 = jnp.sum(f2, axis=0, keepdims=True)
    p1 = 1.0 / (1.0 + jnp.exp(m2 - m1))
    sparse = jnp.where(cnt1 > 1.0, f1 / cnt1,
                       p1 * f1 + (1.0 - p1) * (f2 / cnt2))

    soft = ec / jnp.sum(ec, axis=0, keepdims=True)
    gsum = jnp.sum(sig, axis=0, keepdims=True)

    a = a_ref[0, 0]
    outT = (a * sparse + (1.0 - a) * soft) * gsum
    o_ref[...] = outT.T


def _tgate_kernel(x_hbm, w_ref, b_ref, a_ref, o_ref, xbuf, sems):
    i = pl.program_id(0)

    def copy_in(chunk, slot):
        return pltpu.make_async_copy(
            x_hbm.at[pl.ds(chunk * _CHUNK, _CHUNK), :],
            xbuf.at[slot],
            sems.at[slot])

    @pl.when(i == 0)
    def _():
        for j in range(_NBUF):
            copy_in(j, j).start()

    slot = lax.rem(i, _NBUF)
    copy_in(i, slot).wait()
    _epilogue(xbuf[slot], w_ref, b_ref, a_ref, o_ref)

    @pl.when(i + _NBUF < _STEPS)
    def _():
        copy_in(i + _NBUF, slot).start()


@functools.partial(jax.jit, static_argnames=())
def _tgate(x, w, b, a):
    return pl.pallas_call(
        _tgate_kernel,
        grid=(_STEPS,),
        in_specs=[
            pl.BlockSpec(memory_space=pltpu.MemorySpace.HBM),
            pl.BlockSpec((_D, 3 * _T), lambda i: (0, 0)),
            pl.BlockSpec((3 * _T, 1), lambda i: (0, 0)),
            pl.BlockSpec((1, 1), lambda i: (0, 0)),
        ],
        out_specs=pl.BlockSpec((_CHUNK, _T), lambda i: (i, 0)),
        out_shape=jax.ShapeDtypeStruct((_N, _T), jnp.float32),
        scratch_shapes=[
            pltpu.VMEM((_NBUF, _CHUNK, _D), jnp.float32),
            pltpu.SemaphoreType.DMA((_NBUF,)),
        ],
        compiler_params=pltpu.CompilerParams(
            dimension_semantics=("arbitrary",)),
    )(x, w, b, a)


def kernel(x, Wc, bc, Ws, bs, Wg, bg, alpha):
    w = jnp.concatenate([Wc, -Wg, Ws], axis=1)
    b = jnp.concatenate([bc, -bg, bs], axis=0).reshape(3 * _T, 1)
    a = jax.nn.sigmoid(alpha).reshape(1, 1)
    return _tgate(x, w, b, a)


# speedup vs baseline: 1.0405x; 1.0050x over previous
import functools

import jax
import jax.numpy as jnp
from jax.experimental import pallas as pl
from jax.experimental.pallas import tpu as pltpu

_N = 32768
_D = 768
_T = 8
_BLOCK = 4096


def _tgate_kernel(x_ref, w_ref, b_ref, a_ref, o_ref):
    acc = jnp.dot(x_ref[...], w_ref[...], preferred_element_type=jnp.float32)
    accT = acc.T + b_ref[...]

    e = jnp.exp(accT[0:2 * _T, :])
    ec = e[0:_T, :]
    sig = 1.0 / (1.0 + e[_T:2 * _T, :])
    s = accT[2 * _T:3 * _T, :]

    m1 = jnp.max(s, axis=0, keepdims=True)
    mask1 = s == m1
    f1 = mask1.astype(jnp.float32)
    s2 = jnp.where(mask1, -jnp.inf, s)
    m2 = jnp.max(s2, axis=0, keepdims=True)
    f2 = (s2 == m2).astype(jnp.float32)
    cnt1 = jnp.sum(f1, axis=0, keepdims=True)
    cnt2 = jnp.sum(f2, axis=0, keepdims=True)
    p1 = 1.0 / (1.0 + jnp.exp(m2 - m1))
    sparse = jnp.where(cnt1 > 1.0, f1 / cnt1,
                       p1 * f1 + (1.0 - p1) * (f2 / cnt2))

    soft = ec / jnp.sum(ec, axis=0, keepdims=True)
    gsum = jnp.sum(sig, axis=0, keepdims=True)

    a = a_ref[0, 0]
    outT = (a * sparse + (1.0 - a) * soft) * gsum
    o_ref[...] = outT.T


@functools.partial(jax.jit, static_argnames=())
def _tgate(x, w, b, a):
    grid = (_N // _BLOCK,)
    return pl.pallas_call(
        _tgate_kernel,
        grid=grid,
        in_specs=[
            pl.BlockSpec((_BLOCK, _D), lambda i: (i, 0)),
            pl.BlockSpec((_D, 3 * _T), lambda i: (0, 0)),
            pl.BlockSpec((3 * _T, 1), lambda i: (0, 0)),
            pl.BlockSpec((1, 1), lambda i: (0, 0)),
        ],
        out_specs=pl.BlockSpec((_BLOCK, _T), lambda i: (i, 0)),
        out_shape=jax.ShapeDtypeStruct((_N, _T), jnp.float32),
        compiler_params=pltpu.CompilerParams(
            dimension_semantics=("parallel",)),
    )(x, w, b, a)


def kernel(x, Wc, bc, Ws, bs, Wg, bg, alpha):
    w = jnp.concatenate([Wc, -Wg, Ws], axis=1)
    b = jnp.concatenate([bc, -bg, bs], axis=0).reshape(3 * _T, 1)
    a = jax.nn.sigmoid(alpha).reshape(1, 1)
    return _tgate(x, w, b, a)
